# baseline (speedup 1.0000x reference)
import functools

import jax
import jax.numpy as jnp
from jax import lax
from jax.experimental import pallas as pl
from jax.experimental.pallas import tpu as pltpu
from jax.experimental.pallas import tpu_sc as plsc

_N, _C, _RES, _F = 32, 192, 32, 4
_LH = _RES // _F
_HW = _LH * _LH
_NB = _F * _F
_K = 6
_R = _NB * _N
_INDIM = _K * _C + _HW
_HID = _INDIM
_OC = _C // 10 + 1
_OUTD = _HW * _OC

_SC_NC, _SC_NS = 2, 16
_NW = _SC_NC * _SC_NS
_NIDX = _R * _K
_BPW = _NIDX // _NW
_CP = 256


def _scores_body(xb_ref, b_ref, t_ref):
    a = xb_ref[0]
    b = b_ref[0]
    t_ref[0] = jnp.dot(a, b, preferred_element_type=jnp.float32)


def _run_scores(xb3, bmat):
    return pl.pallas_call(
        _scores_body,
        grid=(_NB,),
        in_specs=[
            pl.BlockSpec((1, _N * _HW, _CP), lambda i: (i, 0, 0)),
            pl.BlockSpec((1, _CP, 8), lambda i: (i, 0, 0)),
        ],
        out_specs=pl.BlockSpec((1, _N * _HW, 8), lambda i: (i, 0, 0)),
        out_shape=jax.ShapeDtypeStruct((_NB, _N * _HW, 8), jnp.float32),
    )(xb3, bmat)


def _select_body(t_ref, mask_ref, gidx_ref):
    bi = pl.program_id(0)
    t = t_ref[0]
    e = jnp.exp(t)
    normed = e / jnp.sum(e, axis=1, keepdims=True)
    pos = lax.broadcasted_iota(jnp.int32, (_N, _HW), 1)
    work = normed
    mask = jnp.zeros((_N, _HW), jnp.float32)
    for _ in range(_K):
        m = jnp.max(work, axis=1, keepdims=True)
        is_max = work == m
        cand = jnp.where(is_max, pos, _HW)
        sel = jnp.min(cand, axis=1, keepdims=True)
        one = pos == sel
        mask = mask + one.astype(jnp.float32)
        work = jnp.where(one, -jnp.inf, work)
    mask_ref[0] = mask
    candp = jnp.where(mask > 0.5, pos, _HW)
    rowbase = (bi * _N + lax.broadcasted_iota(jnp.int32, (_N, 1), 0)) * _HW
    cols = []
    for _ in range(_K):
        mn = jnp.min(candp, axis=1, keepdims=True)
        cols.append(rowbase + mn)
        candp = jnp.where(candp == mn, _HW, candp)
    gidx_ref[0] = jnp.concatenate(cols, axis=1)


def _run_select(t):
    mask, gidx = pl.pallas_call(
        _select_body,
        grid=(_NB,),
        in_specs=[
            pl.BlockSpec((1, _N, _HW), lambda i: (i, 0, 0)),
        ],
        out_specs=[
            pl.BlockSpec((1, _N, _HW), lambda i: (i, 0, 0)),
            pl.BlockSpec((1, _N, _K), lambda i: (i, 0, 0)),
        ],
        out_shape=[
            jax.ShapeDtypeStruct((_NB, _N, _HW), jnp.float32),
            jax.ShapeDtypeStruct((_NB, _N, _K), jnp.int32),
        ],
    )(t)
    return mask, gidx


def _make_sc_gather():
    mesh = plsc.VectorSubcoreMesh(core_axis_name="c", subcore_axis_name="s")

    @functools.partial(
        pl.kernel,
        mesh=mesh,
        out_type=jax.ShapeDtypeStruct((_NIDX, _CP // 2), jnp.int32),
        scratch_types=[
            pltpu.VMEM((_BPW,), jnp.int32),
            pltpu.VMEM((_BPW, _CP // 2), jnp.int32),
            pltpu.SemaphoreType.DMA,
        ],
    )
    def gather_k(table_hbm, idx_hbm, out_hbm, idx_v, rows_v, sem):
        wid = lax.axis_index("s") * _SC_NC + lax.axis_index("c")
        base = wid * _BPW
        pltpu.sync_copy(idx_hbm.at[pl.ds(base, _BPW)], idx_v)
        pltpu.async_copy(table_hbm.at[idx_v], rows_v, sem).wait()
        pltpu.sync_copy(rows_v, out_hbm.at[pl.ds(base, _BPW)])

    return gather_k


_sc_gather_impl = None


def _sc_gather(table, idx):
    global _sc_gather_impl
    if _sc_gather_impl is None:
        _sc_gather_impl = _make_sc_gather()
    return _sc_gather_impl(table, idx)


def _mlp_body(cin_ref, w1t_ref, b1_ref, w2t_ref, b2_ref, out_ref):
    w1t = w1t_ref[...].astype(jnp.bfloat16)
    hid = jnp.dot(cin_ref[...], w1t, preferred_element_type=jnp.float32)
    hid = jnp.maximum(hid + b1_ref[...], 0.0).astype(jnp.bfloat16)
    w2t = w2t_ref[...].astype(jnp.bfloat16)
    out_ref[...] = (
        jnp.dot(hid, w2t, preferred_element_type=jnp.float32)
        + b2_ref[...]
    )


def _run_mlp(cin, W1, b1, W2, b2):
    return pl.pallas_call(
        _mlp_body,
        out_shape=jax.ShapeDtypeStruct((_R, _OUTD), jnp.float32),
    )(cin, W1.T, b1.reshape(1, _HID), W2.T, b2.reshape(1, _OUTD))


def _outconv_body(full_ref, ow_ref, out_ref):
    out_ref[0] = jnp.dot(
        ow_ref[...], full_ref[0], preferred_element_type=jnp.float32
    )


def _run_outconv(full, out_w):
    return pl.pallas_call(
        _outconv_body,
        grid=(_N,),
        in_specs=[
            pl.BlockSpec((1, _OC, _RES * _RES), lambda i: (i, 0, 0)),
            pl.BlockSpec((_C, _OC), lambda i: (0, 0)),
        ],
        out_specs=pl.BlockSpec((1, _C, _RES * _RES), lambda i: (i, 0, 0)),
        out_shape=jax.ShapeDtypeStruct((_N, _C, _RES * _RES), jnp.float32),
    )(full, out_w)


def kernel(x, conv_w, W1, b1, W2, b2, out_w):
    n, c, h, w = x.shape
    xbp = jnp.pad(
        x.reshape(n, c, _F, _LH, _F, _LH)
        .transpose(2, 4, 0, 3, 5, 1)
        .reshape(_NB, _N * _HW, _C)
        .astype(jnp.bfloat16),
        ((0, 0), (0, 0), (0, _CP - _C)),
    )
    bmat = (
        jnp.zeros((_NB, _CP, 8), jnp.float32)
        .at[:, :_C, 0]
        .set(conv_w)
        .astype(jnp.bfloat16)
    )
    t8 = _run_scores(xbp, bmat)
    t = t8[:, :, 0].reshape(_NB, _N, _HW)
    mask, gidx = _run_select(t)

    table = jax.lax.bitcast_convert_type(
        xbp.reshape(_R * _HW, _CP // 2, 2), jnp.int32
    )
    gath_i32 = _sc_gather(table, gidx.reshape(_NIDX))
    gathered = jax.lax.bitcast_convert_type(gath_i32, jnp.bfloat16).reshape(
        _NIDX, _CP
    )

    cin = jnp.concatenate(
        [
            gathered[:, :_C].reshape(_R, _K * _C),
            mask.reshape(_R, _HW).astype(jnp.bfloat16),
        ],
        axis=1,
    )
    rec = _run_mlp(cin, W1, b1, W2, b2)

    full = (
        rec.reshape(_F, _F, _N, _OC, _LH, _LH)
        .transpose(2, 3, 0, 4, 1, 5)
        .reshape(_N, _OC, _RES * _RES)
    )
    out = _run_outconv(full, out_w)
    return out.reshape(_N, _C, _RES, _RES)


# speedup vs baseline: 1.5265x; 1.5265x over previous
import functools

import jax
import jax.numpy as jnp
from jax import lax
from jax.experimental import pallas as pl
from jax.experimental.pallas import tpu as pltpu
from jax.experimental.pallas import tpu_sc as plsc

_N, _C, _RES, _F = 32, 192, 32, 4
_LH = _RES // _F
_HW = _LH * _LH
_NB = _F * _F
_K = 6
_R = _NB * _N
_INDIM = _K * _C + _HW
_HID = _INDIM
_OC = _C // 10 + 1
_OUTD = _HW * _OC

_SC_NC, _SC_NS = 2, 16
_NW = _SC_NC * _SC_NS
_NIDX = _R * _K
_BPW = _NIDX // _NW
_CP = 256


def _scores_body(xb_ref, b_ref, t_ref):
    a = xb_ref[0].astype(jnp.bfloat16)
    b = b_ref[0].astype(jnp.bfloat16)
    t_ref[0] = jnp.dot(a, b, preferred_element_type=jnp.float32)


def _run_scores(xb3, bmat):
    return pl.pallas_call(
        _scores_body,
        grid=(_NB,),
        in_specs=[
            pl.BlockSpec((1, _N * _HW, _CP), lambda i: (i, 0, 0)),
            pl.BlockSpec((1, _CP, 8), lambda i: (i, 0, 0)),
        ],
        out_specs=pl.BlockSpec((1, _N * _HW, 8), lambda i: (i, 0, 0)),
        out_shape=jax.ShapeDtypeStruct((_NB, _N * _HW, 8), jnp.float32),
    )(xb3, bmat)


def _select_body(t_ref, mask_ref, gidx_ref):
    bi = pl.program_id(0)
    t = t_ref[0]
    e = jnp.exp(t)
    normed = e / jnp.sum(e, axis=1, keepdims=True)
    pos = lax.broadcasted_iota(jnp.int32, (_N, _HW), 1)
    work = normed
    mask = jnp.zeros((_N, _HW), jnp.float32)
    for _ in range(_K):
        m = jnp.max(work, axis=1, keepdims=True)
        is_max = work == m
        cand = jnp.where(is_max, pos, _HW)
        sel = jnp.min(cand, axis=1, keepdims=True)
        one = pos == sel
        mask = mask + one.astype(jnp.float32)
        work = jnp.where(one, -jnp.inf, work)
    mask_ref[0] = mask
    candp = jnp.where(mask > 0.5, pos, _HW)
    rowbase = (bi * _N + lax.broadcasted_iota(jnp.int32, (_N, 1), 0)) * _HW
    cols = []
    for _ in range(_K):
        mn = jnp.min(candp, axis=1, keepdims=True)
        cols.append(rowbase + mn)
        candp = jnp.where(candp == mn, _HW, candp)
    gidx_ref[0] = jnp.concatenate(cols, axis=1)


def _run_select(t):
    mask, gidx = pl.pallas_call(
        _select_body,
        grid=(_NB,),
        in_specs=[
            pl.BlockSpec((1, _N, _HW), lambda i: (i, 0, 0)),
        ],
        out_specs=[
            pl.BlockSpec((1, _N, _HW), lambda i: (i, 0, 0)),
            pl.BlockSpec((1, _N, _K), lambda i: (i, 0, 0)),
        ],
        out_shape=[
            jax.ShapeDtypeStruct((_NB, _N, _HW), jnp.float32),
            jax.ShapeDtypeStruct((_NB, _N, _K), jnp.int32),
        ],
    )(t)
    return mask, gidx


def _make_sc_gather():
    mesh = plsc.VectorSubcoreMesh(core_axis_name="c", subcore_axis_name="s")

    @functools.partial(
        pl.kernel,
        mesh=mesh,
        out_type=jax.ShapeDtypeStruct((_NIDX, _CP), jnp.float32),
        scratch_types=[
            pltpu.VMEM((_BPW,), jnp.int32),
            pltpu.VMEM((_BPW, _CP), jnp.float32),
            pltpu.SemaphoreType.DMA,
        ],
    )
    def gather_k(table_hbm, idx_hbm, out_hbm, idx_v, rows_v, sem):
        wid = lax.axis_index("s") * _SC_NC + lax.axis_index("c")
        base = wid * _BPW
        pltpu.sync_copy(idx_hbm.at[pl.ds(base, _BPW)], idx_v)
        pltpu.async_copy(table_hbm.at[idx_v], rows_v, sem).wait()
        pltpu.sync_copy(rows_v, out_hbm.at[pl.ds(base, _BPW)])

    return gather_k


_sc_gather_impl = None


def _sc_gather(table, idx):
    global _sc_gather_impl
    if _sc_gather_impl is None:
        _sc_gather_impl = _make_sc_gather()
    return _sc_gather_impl(table, idx)


def _mlp_body(cin_ref, w1t_ref, b1_ref, w2t_ref, b2_ref, out_ref):
    w1t = w1t_ref[...].astype(jnp.bfloat16)
    hid = jnp.dot(cin_ref[...], w1t, preferred_element_type=jnp.float32)
    hid = jnp.maximum(hid + b1_ref[...], 0.0).astype(jnp.bfloat16)
    w2t = w2t_ref[...].astype(jnp.bfloat16)
    out_ref[...] = (
        jnp.dot(hid, w2t, preferred_element_type=jnp.float32)
        + b2_ref[...]
    )


def _run_mlp(cin, W1, b1, W2, b2):
    return pl.pallas_call(
        _mlp_body,
        out_shape=jax.ShapeDtypeStruct((_R, _OUTD), jnp.float32),
    )(cin, W1.T, b1.reshape(1, _HID), W2.T, b2.reshape(1, _OUTD))


def _outconv_body(full_ref, ow_ref, out_ref):
    out_ref[0] = jnp.dot(
        ow_ref[...], full_ref[0], preferred_element_type=jnp.float32
    )


def _run_outconv(full, out_w):
    return pl.pallas_call(
        _outconv_body,
        grid=(_N,),
        in_specs=[
            pl.BlockSpec((1, _OC, _RES * _RES), lambda i: (i, 0, 0)),
            pl.BlockSpec((_C, _OC), lambda i: (0, 0)),
        ],
        out_specs=pl.BlockSpec((1, _C, _RES * _RES), lambda i: (i, 0, 0)),
        out_shape=jax.ShapeDtypeStruct((_N, _C, _RES * _RES), jnp.float32),
    )(full, out_w)


def kernel(x, conv_w, W1, b1, W2, b2, out_w):
    n, c, h, w = x.shape
    xbp = (
        jnp.zeros((_NB, _N * _HW, _CP), jnp.float32)
        .at[:, :, :_C]
        .set(
            x.reshape(n, c, _F, _LH, _F, _LH)
            .transpose(2, 4, 0, 3, 5, 1)
            .reshape(_NB, _N * _HW, _C)
        )
    )
    bmat = jnp.zeros((_NB, _CP, 8), jnp.float32).at[:, :_C, 0].set(conv_w)
    t8 = _run_scores(xbp, bmat)
    t = t8[:, :, 0].reshape(_NB, _N, _HW)
    mask, gidx = _run_select(t)

    table = xbp.reshape(_R * _HW, _CP)
    gathered = _sc_gather(table, gidx.reshape(_NIDX))

    cin = jnp.concatenate(
        [
            gathered[:, :_C].reshape(_R, _K * _C),
            mask.reshape(_R, _HW),
        ],
        axis=1,
    ).astype(jnp.bfloat16)
    rec = _run_mlp(cin, W1, b1, W2, b2)

    full = (
        rec.reshape(_F, _F, _N, _OC, _LH, _LH)
        .transpose(2, 3, 0, 4, 1, 5)
        .reshape(_N, _OC, _RES * _RES)
    )
    out = _run_outconv(full, out_w)
    return out.reshape(_N, _C, _RES, _RES)


# speedup vs baseline: 2.5081x; 1.6430x over previous
import functools

import jax
import jax.numpy as jnp
from jax import lax
from jax.experimental import pallas as pl
from jax.experimental.pallas import tpu as pltpu
from jax.experimental.pallas import tpu_sc as plsc

_N, _C, _RES, _F = 32, 192, 32, 4
_LH = _RES // _F
_HW = _LH * _LH
_NB = _F * _F
_K = 6
_R = _NB * _N
_INDIM = _K * _C + _HW
_HID = _INDIM
_OC = _C // 10 + 1
_OUTD = _HW * _OC
_NROWS = _N * _RES * _RES

_SC_NC, _SC_NS = 2, 16
_NW = _SC_NC * _SC_NS
_NIDX = _R * _K
_BPW = _NIDX // _NW


def _scores_body(xr_ref, b_ref, t_ref):
    a = xr_ref[0].astype(jnp.bfloat16)
    b = b_ref[...].astype(jnp.bfloat16)
    t_ref[0] = jnp.dot(a, b, preferred_element_type=jnp.float32)


def _run_scores(xrows, cwt):
    xr3 = xrows.reshape(_NB, _NROWS // _NB, _C)
    return pl.pallas_call(
        _scores_body,
        grid=(_NB,),
        in_specs=[
            pl.BlockSpec((1, _NROWS // _NB, _C), lambda i: (i, 0, 0)),
            pl.BlockSpec((_C, _NB), lambda i: (0, 0)),
        ],
        out_specs=pl.BlockSpec((1, _NROWS // _NB, _NB), lambda i: (i, 0, 0)),
        out_shape=jax.ShapeDtypeStruct((_NB, _NROWS // _NB, _NB), jnp.float32),
    )(xr3, cwt)


def _select_body(t_ref, mask_ref, gidx_ref):
    t = t_ref[...]
    e = jnp.exp(t)
    normed = e / jnp.sum(e, axis=1, keepdims=True)
    pos = lax.broadcasted_iota(jnp.int32, (_R, _HW), 1)
    work = normed
    mask = jnp.zeros((_R, _HW), jnp.float32)
    for _ in range(_K):
        m = jnp.max(work, axis=1, keepdims=True)
        is_max = work == m
        cand = jnp.where(is_max, pos, _HW)
        sel = jnp.min(cand, axis=1, keepdims=True)
        one = pos == sel
        mask = mask + one.astype(jnp.float32)
        work = jnp.where(one, -jnp.inf, work)
    mask_ref[...] = mask
    r = lax.broadcasted_iota(jnp.int32, (_R, 1), 0)
    n = r & (_N - 1)
    bi = r >> 5
    rr = bi >> 2
    ss = bi & 3
    base = n * (_RES * _RES) + rr * (_LH * _RES) + ss * _LH
    candp = jnp.where(mask > 0.5, pos, _HW)
    cols = []
    for _ in range(_K):
        p = jnp.min(candp, axis=1, keepdims=True)
        cols.append(base + (p >> 3) * _RES + (p & 7))
        candp = jnp.where(candp == p, _HW, candp)
    gidx_ref[...] = jnp.concatenate(cols, axis=1)


def _run_select(t):
    mask, gidx = pl.pallas_call(
        _select_body,
        out_shape=[
            jax.ShapeDtypeStruct((_R, _HW), jnp.float32),
            jax.ShapeDtypeStruct((_R, _K), jnp.int32),
        ],
    )(t)
    return mask, gidx


def _make_sc_gather():
    mesh = plsc.VectorSubcoreMesh(core_axis_name="c", subcore_axis_name="s")

    @functools.partial(
        pl.kernel,
        mesh=mesh,
        out_type=jax.ShapeDtypeStruct((_NIDX, _C), jnp.float32),
        scratch_types=[
            pltpu.VMEM((_BPW,), jnp.int32),
            pltpu.VMEM((_BPW, _C), jnp.float32),
            pltpu.SemaphoreType.DMA,
        ],
        compiler_params=pltpu.CompilerParams(use_tc_tiling_on_sc=False),
    )
    def gather_k(table_hbm, idx_hbm, out_hbm, idx_v, rows_v, sem):
        wid = lax.axis_index("s") * _SC_NC + lax.axis_index("c")
        base = wid * _BPW
        pltpu.sync_copy(idx_hbm.at[pl.ds(base, _BPW)], idx_v)
        pltpu.async_copy(table_hbm.at[idx_v], rows_v, sem).wait()
        pltpu.sync_copy(rows_v, out_hbm.at[pl.ds(base, _BPW)])

    return gather_k


_sc_gather_impl = None


def _sc_gather(table, idx):
    global _sc_gather_impl
    if _sc_gather_impl is None:
        _sc_gather_impl = _make_sc_gather()
    return _sc_gather_impl(table, idx)


def _mlp_body(cin_ref, w1_ref, b1_ref, w2_ref, b2_ref, out_ref):
    w1 = w1_ref[...].astype(jnp.bfloat16)
    hid = lax.dot_general(
        cin_ref[...], w1,
        (((1,), (1,)), ((), ())),
        preferred_element_type=jnp.float32,
    )
    hid = jnp.maximum(hid + b1_ref[...], 0.0).astype(jnp.bfloat16)
    w2 = w2_ref[...].astype(jnp.bfloat16)
    out_ref[...] = (
        lax.dot_general(
            hid, w2,
            (((1,), (1,)), ((), ())),
            preferred_element_type=jnp.float32,
        )
        + b2_ref[...]
    )


def _run_mlp(cin, W1, b1, W2, b2):
    return pl.pallas_call(
        _mlp_body,
        out_shape=jax.ShapeDtypeStruct((_R, _OUTD), jnp.float32),
    )(cin, W1, b1.reshape(1, _HID), W2, b2.reshape(1, _OUTD))


def _outconv_body(full_ref, ow_ref, out_ref):
    out_ref[0] = jnp.dot(
        full_ref[0], ow_ref[...], preferred_element_type=jnp.float32
    )


def _run_outconv(full_r, owt):
    fr3 = full_r.reshape(8, _NROWS // 8, _OC)
    return pl.pallas_call(
        _outconv_body,
        grid=(8,),
        in_specs=[
            pl.BlockSpec((1, _NROWS // 8, _OC), lambda i: (i, 0, 0)),
            pl.BlockSpec((_OC, _C), lambda i: (0, 0)),
        ],
        out_specs=pl.BlockSpec((1, _NROWS // 8, _C), lambda i: (i, 0, 0)),
        out_shape=jax.ShapeDtypeStruct((8, _NROWS // 8, _C), jnp.float32),
    )(fr3, owt)


def kernel(x, conv_w, W1, b1, W2, b2, out_w):
    n, c, h, w = x.shape
    xrows = x.transpose(0, 2, 3, 1).reshape(_NROWS, _C)

    t_all = _run_scores(xrows, conv_w.T)
    t6 = t_all.reshape(_N, _F, _LH, _F, _LH, _NB)
    t6 = t6.transpose(1, 3, 0, 2, 4, 5)
    biidx = jnp.arange(_NB, dtype=jnp.int32).reshape(_F, _F, 1, 1, 1, 1)
    t_sel = jnp.take_along_axis(t6, biidx, axis=5)[..., 0].reshape(_R, _HW)

    mask, gidx = _run_select(t_sel)

    gathered = _sc_gather(xrows, gidx.reshape(_NIDX))

    cin = jnp.concatenate(
        [gathered.reshape(_R, _K * _C), mask], axis=1
    ).astype(jnp.bfloat16)
    rec = _run_mlp(cin, W1, b1, W2, b2)

    full_r = (
        rec.reshape(_F, _F, _N, _OC, _LH, _LH)
        .transpose(2, 0, 4, 1, 5, 3)
        .reshape(_NROWS, _OC)
    )
    out_r = _run_outconv(full_r, out_w.T)
    return (
        out_r.reshape(_N, _RES, _RES, _C).transpose(0, 3, 1, 2)
    )
